# Initial kernel scaffold; baseline (speedup 1.0000x reference)
import jax
import jax.numpy as jnp
from jax.experimental import pallas as pl


def kernel(input_tensor, weight):
    raise NotImplementedError("write your pallas kernel here")


# speedup vs baseline: 3.5416x; 3.5416x over previous
import functools

import jax
import jax.numpy as jnp
from jax import lax
from jax.experimental import pallas as pl
from jax.experimental.pallas import tpu as pltpu
from jax.experimental.pallas import tpu_sc as plsc

BATCH = 4096
SEQ = 200
EMB = 64

NC, NS = 2, 16
NW = NC * NS
B = BATCH * SEQ
CHUNK = 128
BPW = B // NW
NCHUNK = BPW // CHUNK


def _emb_body(idx_hbm, tab_hbm, out_hbm, idx_v, rows_v, sem):
    w = lax.axis_index("s") * NC + lax.axis_index("c")
    base = w * BPW
    pltpu.sync_copy(idx_hbm.at[w], idx_v)

    @pl.loop(0, NCHUNK)
    def _(j):
        pltpu.async_copy(tab_hbm.at[idx_v.at[j]], rows_v, sem).wait()
        pltpu.sync_copy(rows_v, out_hbm.at[pl.ds(base + j * CHUNK, CHUNK)])


@jax.jit
def kernel(input_tensor, weight):
    idx = input_tensor.reshape(NW, NCHUNK, CHUNK).astype(jnp.int32)
    mesh = plsc.VectorSubcoreMesh(
        core_axis_name="c", subcore_axis_name="s", num_cores=NC, num_subcores=NS
    )
    out = pl.kernel(
        _emb_body,
        out_type=jax.ShapeDtypeStruct((B, EMB), jnp.float32),
        mesh=mesh,
        scratch_types=[
            pltpu.VMEM((NCHUNK, CHUNK), jnp.int32),
            pltpu.VMEM((CHUNK, EMB), jnp.float32),
            pltpu.SemaphoreType.DMA,
        ],
        compiler_params=pltpu.CompilerParams(use_tc_tiling_on_sc=False),
    )(idx, weight)
    return out.reshape(BATCH, SEQ, EMB)


# speedup vs baseline: 4.2659x; 1.2045x over previous
import functools

import jax
import jax.numpy as jnp
from jax import lax
from jax.experimental import pallas as pl
from jax.experimental.pallas import tpu as pltpu
from jax.experimental.pallas import tpu_sc as plsc

BATCH = 4096
SEQ = 200
EMB = 64

NC, NS = 2, 16
NW = NC * NS
B = BATCH * SEQ
CHUNK = 128
BPW = B // NW
NCHUNK = BPW // CHUNK
K = 4
GROUP = K * CHUNK
NGROUP = NCHUNK // K
NBUF = 2


def _emb_body(idx_hbm, tab_hbm, out_hbm, idx_v, rows_v, sg0, sg1, so0, so1):
    w = lax.axis_index("s") * NC + lax.axis_index("c")
    base = w * BPW
    sem_g = [sg0, sg1]
    sem_o = [so0, so1]

    pltpu.sync_copy(idx_hbm.at[w], idx_v)

    def fire_gathers(b, gid):
        for k in range(K):
            pltpu.async_copy(
                tab_hbm.at[idx_v.at[gid * K + k]],
                rows_v.at[b, pl.ds(k * CHUNK, CHUNK)],
                sem_g[b],
            )

    def drain_gathers(b):
        pltpu.make_async_copy(
            out_hbm.at[pl.ds(base, GROUP)], rows_v.at[b], sem_g[b]
        ).wait()

    def fire_out(b, gid):
        pltpu.async_copy(
            rows_v.at[b], out_hbm.at[pl.ds(base + gid * GROUP, GROUP)], sem_o[b]
        )

    def drain_out(b):
        pltpu.make_async_copy(
            out_hbm.at[pl.ds(base, GROUP)], rows_v.at[b], sem_o[b]
        ).wait()

    @pl.loop(0, NGROUP, step=NBUF)
    def _(g):
        for b in range(NBUF):
            gid = g + b

            @pl.when(gid >= NBUF)
            def _():
                drain_out(b)

            fire_gathers(b, gid)

            bp = (b - 1) % NBUF

            @pl.when(gid >= 1)
            def _():
                drain_gathers(bp)
                fire_out(bp, gid - 1)

    last = (NGROUP - 1) % NBUF
    drain_gathers(last)
    fire_out(last, NGROUP - 1)
    for b in range(NBUF):
        drain_out(b)


@jax.jit
def kernel(input_tensor, weight):
    idx = input_tensor.reshape(NW, NCHUNK, CHUNK).astype(jnp.int32)
    mesh = plsc.VectorSubcoreMesh(
        core_axis_name="c", subcore_axis_name="s", num_cores=NC, num_subcores=NS
    )
    out = pl.kernel(
        _emb_body,
        out_type=jax.ShapeDtypeStruct((B, EMB), jnp.float32),
        mesh=mesh,
        scratch_types=[
            pltpu.VMEM((NCHUNK, CHUNK), jnp.int32),
            pltpu.VMEM((NBUF, GROUP, EMB), jnp.float32),
            pltpu.SemaphoreType.DMA,
            pltpu.SemaphoreType.DMA,
            pltpu.SemaphoreType.DMA,
            pltpu.SemaphoreType.DMA,
        ],
        compiler_params=pltpu.CompilerParams(use_tc_tiling_on_sc=False),
    )(idx, weight)
    return out.reshape(BATCH, SEQ, EMB)


# speedup vs baseline: 4.2703x; 1.0010x over previous
import functools

import jax
import jax.numpy as jnp
from jax import lax
from jax.experimental import pallas as pl
from jax.experimental.pallas import tpu as pltpu
from jax.experimental.pallas import tpu_sc as plsc

BATCH = 4096
SEQ = 200
EMB = 64

NC, NS = 2, 16
NW = NC * NS
B = BATCH * SEQ
CHUNK = 256
BPW = B // NW
NCHUNK = BPW // CHUNK
K = 2
GROUP = K * CHUNK
NGROUP = NCHUNK // K
NBUF = 2


def _emb_body(idx_hbm, tab_hbm, out_hbm, idx_v, rows_v, sg0, sg1, so0, so1):
    w = lax.axis_index("s") * NC + lax.axis_index("c")
    base = w * BPW
    sem_g = [sg0, sg1]
    sem_o = [so0, so1]

    pltpu.sync_copy(idx_hbm.at[w], idx_v)

    def fire_gathers(b, gid):
        for k in range(K):
            pltpu.async_copy(
                tab_hbm.at[idx_v.at[gid * K + k]],
                rows_v.at[b, pl.ds(k * CHUNK, CHUNK)],
                sem_g[b],
            )

    def drain_gathers(b):
        pltpu.make_async_copy(
            out_hbm.at[pl.ds(base, GROUP)], rows_v.at[b], sem_g[b]
        ).wait()

    def fire_out(b, gid):
        pltpu.async_copy(
            rows_v.at[b], out_hbm.at[pl.ds(base + gid * GROUP, GROUP)], sem_o[b]
        )

    def drain_out(b):
        pltpu.make_async_copy(
            out_hbm.at[pl.ds(base, GROUP)], rows_v.at[b], sem_o[b]
        ).wait()

    @pl.loop(0, NGROUP, step=NBUF)
    def _(g):
        for b in range(NBUF):
            gid = g + b

            @pl.when(gid >= NBUF)
            def _():
                drain_out(b)

            fire_gathers(b, gid)

            bp = (b - 1) % NBUF

            @pl.when(gid >= 1)
            def _():
                drain_gathers(bp)
                fire_out(bp, gid - 1)

    last = (NGROUP - 1) % NBUF
    drain_gathers(last)
    fire_out(last, NGROUP - 1)
    for b in range(NBUF):
        drain_out(b)


@jax.jit
def kernel(input_tensor, weight):
    idx = input_tensor.reshape(NW, NCHUNK, CHUNK).astype(jnp.int32)
    mesh = plsc.VectorSubcoreMesh(
        core_axis_name="c", subcore_axis_name="s", num_cores=NC, num_subcores=NS
    )
    out = pl.kernel(
        _emb_body,
        out_type=jax.ShapeDtypeStruct((B, EMB), jnp.float32),
        mesh=mesh,
        scratch_types=[
            pltpu.VMEM((NCHUNK, CHUNK), jnp.int32),
            pltpu.VMEM((NBUF, GROUP, EMB), jnp.float32),
            pltpu.SemaphoreType.DMA,
            pltpu.SemaphoreType.DMA,
            pltpu.SemaphoreType.DMA,
            pltpu.SemaphoreType.DMA,
        ],
        compiler_params=pltpu.CompilerParams(use_tc_tiling_on_sc=False),
    )(idx, weight)
    return out.reshape(BATCH, SEQ, EMB)
